# Initial kernel scaffold; baseline (speedup 1.0000x reference)
import functools
from functools import partial

import numpy as np
import jax
import jax.numpy as jnp
from jax import lax
from jax.experimental import pallas as pl
from jax.experimental.pallas import tpu as pltpu
from jax.experimental.pallas import tpu_sc as plsc

F32 = jnp.float32
I32 = jnp.int32

_N_NET, _E_NET = 10000, 320000
_N_DAG, _E_DAG = 1000, 8000
_NP_NET, _EP_NET = 10240, 157 * 2048
_NP_DAG, _EP_DAG = 1024, 4 * 2048
_EMB = 256


def _sc_norm(Npad, EP):
    ns = Npad // 16
    ept = EP // 16
    nb1 = ept // 128
    ept2 = EP // 32
    nb2 = ept2 // 64
    mesh = plsc.VectorSubcoreMesh(core_axis_name="c", subcore_axis_name="s")

    @partial(pl.kernel, mesh=mesh,
             out_type=[jax.ShapeDtypeStruct((EP,), F32),
                       jax.ShapeDtypeStruct((Npad,), F32)],
             scratch_types=[
                 pltpu.VMEM((128,), I32),
                 pltpu.VMEM((128,), F32),
                 pltpu.VMEM((64,), I32),
                 pltpu.VMEM((64,), I32),
                 pltpu.VMEM((64,), F32),
                 pltpu.VMEM((64,), F32),
                 pltpu.VMEM((ns,), F32),
                 pltpu.VMEM((ns,), F32),
                 pltpu.VMEM((Npad,), F32),
                 pltpu.VMEM_SHARED((Npad,), F32),
                 pltpu.VMEM_SHARED((Npad,), F32),
             ])
    def k(src_h, dst_h, ew_h, norm_o, d2_o,
          idx_v, w_v, src_v, dst_v, ew_v, out_v, node_v, d2_v, dinvt,
          degacc, dinvsh):
        c = lax.axis_index("c")
        s = lax.axis_index("s")

        def fill1(i, _):
            node_v[pl.ds(i * 16, 16)] = jnp.ones((16,), F32)
            return 0
        lax.fori_loop(0, ns // 16, fill1, 0)
        pltpu.sync_copy(node_v, degacc.at[pl.ds(s * ns, ns)])
        plsc.subcore_barrier()

        def dbatch(b, _):
            base = s * ept + b * 128
            pltpu.sync_copy(dst_h.at[pl.ds(base, 128)], idx_v)
            pltpu.sync_copy(ew_h.at[pl.ds(base, 128)], w_v)
            pltpu.sync_copy(w_v, degacc.at[idx_v], add=True)
            return 0
        lax.fori_loop(0, nb1, dbatch, 0)
        plsc.subcore_barrier()

        pltpu.sync_copy(degacc.at[pl.ds(s * ns, ns)], node_v)

        def rsq(g, _):
            x = node_v[pl.ds(g * 16, 16)]
            i = plsc.bitcast(x, I32)
            i = jnp.int32(0x5F3759DF) - lax.shift_right_arithmetic(i, 1)
            y = plsc.bitcast(i, F32)
            xh = x * 0.5
            y = y * (1.5 - xh * y * y)
            y = y * (1.5 - xh * y * y)
            y = y * (1.5 - xh * y * y)
            node_v[pl.ds(g * 16, 16)] = y
            d2_v[pl.ds(g * 16, 16)] = y * y
            return 0
        lax.fori_loop(0, ns // 16, rsq, 0)
        pltpu.sync_copy(node_v, dinvsh.at[pl.ds(s * ns, ns)])

        @pl.when(c == 0)
        def _():
            pltpu.sync_copy(d2_v, d2_o.at[pl.ds(s * ns, ns)])

        plsc.subcore_barrier()
        pltpu.sync_copy(dinvsh, dinvt)

        def nbatch(b, _):
            base = c * (EP // 2) + s * ept2 + b * 64
            pltpu.sync_copy(src_h.at[pl.ds(base, 64)], src_v)
            pltpu.sync_copy(dst_h.at[pl.ds(base, 64)], dst_v)
            pltpu.sync_copy(ew_h.at[pl.ds(base, 64)], ew_v)

            def grp(g, _):
                ss = src_v[pl.ds(g * 16, 16)]
                dd = dst_v[pl.ds(g * 16, 16)]
                ww = ew_v[pl.ds(g * 16, 16)]
                a = plsc.load_gather(dinvt, [ss])
                bb = plsc.load_gather(dinvt, [dd])
                out_v[pl.ds(g * 16, 16)] = a * ww * bb
                return 0
            lax.fori_loop(0, 4, grp, 0)
            pltpu.sync_copy(out_v, norm_o.at[pl.ds(base, 64)])
            return 0
        lax.fori_loop(0, nb2, nbatch, 0)

    return k


def _sc_scatter(Npad, EP, Wh):
    rpt = Npad // 16
    zc = min(64, rpt)
    nz = rpt // zc
    wc = min(128, rpt)
    nw = rpt // wc
    ept = EP // 16
    nb = ept // 128
    ng = Wh // 16
    mesh = plsc.VectorSubcoreMesh(core_axis_name="c", subcore_axis_name="s")

    @partial(pl.kernel, mesh=mesh,
             out_type=[jax.ShapeDtypeStruct((Npad, Wh), F32),
                       jax.ShapeDtypeStruct((Npad, Wh), F32)],
             scratch_types=[
                 pltpu.VMEM((128,), I32),
                 pltpu.VMEM((128,), I32),
                 pltpu.VMEM((128,), F32),
                 pltpu.VMEM((128, Wh), F32),
                 pltpu.VMEM((zc, Wh), F32),
                 pltpu.VMEM_SHARED((Npad, Wh), F32),
                 pltpu.SemaphoreType.DMA,
             ])
    def k(xa_h, xb_h, src_h, dst_h, norm_h, oa, ob,
          src_v, dst_v, norm_v, rows_v, zero_v, acc, sem):
        c = lax.axis_index("c")
        s = lax.axis_index("s")

        def zfill(i, _):
            r = i // ng
            j = i % ng
            zero_v[r, pl.ds(j * 16, 16)] = jnp.zeros((16,), F32)
            return 0
        lax.fori_loop(0, zc * ng, zfill, 0)

        def zcopy(i, _):
            pltpu.sync_copy(zero_v, acc.at[pl.ds(s * rpt + i * zc, zc)])
            return 0
        lax.fori_loop(0, nz, zcopy, 0)
        plsc.subcore_barrier()

        def batch(b, _):
            base = s * ept + b * 128
            pltpu.sync_copy(src_h.at[pl.ds(base, 128)], src_v)
            pltpu.sync_copy(dst_h.at[pl.ds(base, 128)], dst_v)
            pltpu.sync_copy(norm_h.at[pl.ds(base, 128)], norm_v)

            @pl.when(c == 0)
            def _():
                pltpu.async_copy(xa_h.at[src_v], rows_v, sem).wait()

            @pl.when(c == 1)
            def _():
                pltpu.async_copy(xb_h.at[src_v], rows_v, sem).wait()

            def scale(r, _):
                bc = plsc.load_gather(norm_v, [jnp.zeros((16,), I32) + r])
                for j in range(ng):
                    rows_v[r, pl.ds(j * 16, 16)] = rows_v[r, pl.ds(j * 16, 16)] * bc
                return 0
            lax.fori_loop(0, 128, scale, 0)
            pltpu.sync_copy(rows_v, acc.at[dst_v], add=True)
            return 0
        lax.fori_loop(0, nb, batch, 0)
        plsc.subcore_barrier()

        def wout(i, _):
            r0 = s * rpt + i * wc
            pltpu.sync_copy(acc.at[pl.ds(r0, wc)], rows_v.at[pl.ds(0, wc)])

            @pl.when(c == 0)
            def _():
                pltpu.sync_copy(rows_v.at[pl.ds(0, wc)], oa.at[pl.ds(r0, wc)])

            @pl.when(c == 1)
            def _():
                pltpu.sync_copy(rows_v.at[pl.ds(0, wc)], ob.at[pl.ds(r0, wc)])
            return 0
        lax.fori_loop(0, nw, wout, 0)

    return k


def _tc_layer(N, BN, Din):
    G = N // BN
    Dh = Din // 2

    def body(sa, sb, x, d2, W0, b0, W1a, W1b, pa, pb):
        ax = jnp.concatenate([sa[...], sb[...]], axis=1) + d2[...] * x[...]
        t = jnp.maximum(
            jnp.dot(ax, W0[...], preferred_element_type=F32) + b0[...], 0.0)
        pa[...] = jnp.dot(t, W1a[...], preferred_element_type=F32)
        pb[...] = jnp.dot(t, W1b[...], preferred_element_type=F32)

    return pl.pallas_call(
        body,
        grid=(G,),
        in_specs=[
            pl.BlockSpec((BN, Dh), lambda i: (i, 0)),
            pl.BlockSpec((BN, Dh), lambda i: (i, 0)),
            pl.BlockSpec((BN, Din), lambda i: (i, 0)),
            pl.BlockSpec((BN, 1), lambda i: (i, 0)),
            pl.BlockSpec((Din, _EMB), lambda i: (0, 0)),
            pl.BlockSpec((1, _EMB), lambda i: (0, 0)),
            pl.BlockSpec((_EMB, _EMB // 2), lambda i: (0, 0)),
            pl.BlockSpec((_EMB, _EMB // 2), lambda i: (0, 0)),
        ],
        out_specs=[
            pl.BlockSpec((BN, _EMB // 2), lambda i: (i, 0)),
            pl.BlockSpec((BN, _EMB // 2), lambda i: (i, 0)),
        ],
        out_shape=[
            jax.ShapeDtypeStruct((N, _EMB // 2), F32),
            jax.ShapeDtypeStruct((N, _EMB // 2), F32),
        ],
    )


def _tc_mean(N, BN):
    G = N // BN
    Dh = _EMB // 2

    def body(sa, sb, pa, pb, d2, b1, out):
        i = pl.program_id(0)
        e = jnp.concatenate([sa[...], sb[...]], axis=1) \
            + d2[...] * jnp.concatenate([pa[...], pb[...]], axis=1) + b1[...]
        e = jnp.maximum(e, 0.0)
        ss = jnp.sum(e * e, axis=1, keepdims=True)
        w = 1.0 / jnp.maximum(jnp.sqrt(ss), 1e-12)
        contrib = jnp.sum(w * e, axis=0, keepdims=True)

        @pl.when(i == 0)
        def _():
            out[...] = jnp.zeros_like(out)
        out[...] += contrib

    return pl.pallas_call(
        body,
        grid=(G,),
        in_specs=[
            pl.BlockSpec((BN, Dh), lambda i: (i, 0)),
            pl.BlockSpec((BN, Dh), lambda i: (i, 0)),
            pl.BlockSpec((BN, Dh), lambda i: (i, 0)),
            pl.BlockSpec((BN, Dh), lambda i: (i, 0)),
            pl.BlockSpec((BN, 1), lambda i: (i, 0)),
            pl.BlockSpec((1, _EMB), lambda i: (0, 0)),
        ],
        out_specs=pl.BlockSpec((1, _EMB), lambda i: (0, 0)),
        out_shape=jax.ShapeDtypeStruct((1, _EMB), F32),
    )


def _mish(x):
    return x * jnp.tanh(jax.nn.softplus(x))


def _tc_final(accn, accd, time2, act, Wt1, bt1, Wt2, bt2,
              Wf1, bf1, Wf2, bf2, Wf3, bf3):
    def body(an, ad, tm, ac, wt1, bt1_, wt2, bt2_, wf1, bf1_, wf2, bf2_,
             wf3, bf3_, out):
        hyb = an[...] * (1.0 / _N_NET) + ad[...] * (1.0 / _N_DAG)
        freqs = jnp.exp(lax.broadcasted_iota(F32, (1, 16), 1)
                        * jnp.float32(-np.log(10000.0) / 15.0))
        e = tm[...] * freqs
        pe = jnp.concatenate([jnp.sin(e), jnp.cos(e)], axis=1)
        temb = jnp.dot(_mish(jnp.dot(pe, wt1[...], preferred_element_type=F32)
                             + bt1_[...]),
                       wt2[...], preferred_element_type=F32) + bt2_[...]
        fi = jnp.concatenate([hyb, temb, ac[...]], axis=1)
        z = _mish(jnp.dot(fi, wf1[...], preferred_element_type=F32) + bf1_[...])
        z = jnp.dot(z, wf2[...], preferred_element_type=F32) + bf2_[...]
        out[...] = jnp.dot(z, wf3[...], preferred_element_type=F32) + bf3_[...]

    return pl.pallas_call(
        body,
        out_shape=jax.ShapeDtypeStruct((1, bf3.shape[0]), F32),
    )(accn, accd, time2, act, Wt1, bt1, Wt2, bt2, Wf1, bf1, Wf2, bf2, Wf3, bf3)


def _pad1(x, n):
    pad = n - x.shape[0]
    return jnp.concatenate([x, jnp.zeros((pad,), x.dtype)])


def _gcn_stack(x, edge_index, edge_weight, W0, b0, W1, b1,
               N, Npad, EP, BN):
    Din = x.shape[1]
    Dh = Din // 2
    src = _pad1(edge_index[0], EP)
    dst = _pad1(edge_index[1], EP)
    ew = _pad1(edge_weight, EP)

    norm, d2 = _sc_norm(Npad, EP)(src, dst, ew)
    d2c = d2[:N, None]

    xa = x[:, :Dh]
    xb = x[:, Dh:]
    s0a, s0b = _sc_scatter(Npad, EP, Dh)(xa, xb, src, dst, norm)
    s0a, s0b = s0a[:N], s0b[:N]

    W1a, W1b = W1[:, :_EMB // 2], W1[:, _EMB // 2:]
    p1a, p1b = _tc_layer(N, BN, Din)(
        s0a, s0b, x, d2c, W0, b0[None, :], W1a, W1b)

    s1a, s1b = _sc_scatter(Npad, EP, _EMB // 2)(p1a, p1b, src, dst, norm)
    s1a, s1b = s1a[:N], s1b[:N]

    return _tc_mean(N, BN)(s1a, s1b, p1a, p1b, d2c, b1[None, :])


def kernel(action, time, net_feature, net_edge_index, net_edge_weight,
           dag_feature, dag_edge_index, dag_edge_weight, batch_size,
           Wn0, bn0, Wn1, bn1, Wd0, bd0, Wd1, bd1,
           Wt1, bt1, Wt2, bt2, Wf1, bf1, Wf2, bf2, Wf3, bf3):
    accn = _gcn_stack(net_feature, net_edge_index, net_edge_weight,
                      Wn0, bn0, Wn1, bn1, _N_NET, _NP_NET, _EP_NET, 1000)
    accd = _gcn_stack(dag_feature, dag_edge_index, dag_edge_weight,
                      Wd0, bd0, Wd1, bd1, _N_DAG, _NP_DAG, _EP_DAG, 1000)
    return _tc_final(accn, accd, time[:, None], action,
                     Wt1, bt1[None, :], Wt2, bt2[None, :],
                     Wf1, bf1[None, :], Wf2, bf2[None, :],
                     Wf3, bf3[None, :])


# speedup vs baseline: 6.4751x; 6.4751x over previous
import functools
from functools import partial

import numpy as np
import jax
import jax.numpy as jnp
from jax import lax
from jax.experimental import pallas as pl
from jax.experimental.pallas import tpu as pltpu
from jax.experimental.pallas import tpu_sc as plsc

F32 = jnp.float32
I32 = jnp.int32

_N_NET, _E_NET = 10000, 320000
_N_DAG, _E_DAG = 1000, 8000
_NP_NET, _EP_NET = 10240, 157 * 2048
_NP_DAG, _EP_DAG = 1024, 4 * 2048
_EMB = 256


def _sc_norm(Npad, EP):
    ns = Npad // 16
    ept = EP // 16
    nb1 = ept // 128
    ept2 = EP // 32
    nb2 = ept2 // 64
    mesh = plsc.VectorSubcoreMesh(core_axis_name="c", subcore_axis_name="s")

    @partial(pl.kernel, mesh=mesh,
             compiler_params=pltpu.CompilerParams(needs_layout_passes=False, use_tc_tiling_on_sc=False),
             out_type=[jax.ShapeDtypeStruct((EP,), F32),
                       jax.ShapeDtypeStruct((Npad,), F32)],
             scratch_types=[
                 pltpu.VMEM((128,), I32),
                 pltpu.VMEM((128,), F32),
                 pltpu.VMEM((64,), I32),
                 pltpu.VMEM((64,), I32),
                 pltpu.VMEM((64,), F32),
                 pltpu.VMEM((64,), F32),
                 pltpu.VMEM((ns,), F32),
                 pltpu.VMEM((ns,), F32),
                 pltpu.VMEM((Npad,), F32),
                 pltpu.VMEM_SHARED((Npad,), F32),
                 pltpu.VMEM_SHARED((Npad,), F32),
             ])
    def k(src_h, dst_h, ew_h, norm_o, d2_o,
          idx_v, w_v, src_v, dst_v, ew_v, out_v, node_v, d2_v, dinvt,
          degacc, dinvsh):
        c = lax.axis_index("c")
        s = lax.axis_index("s")

        def fill1(i, _):
            node_v[pl.ds(i * 16, 16)] = jnp.ones((16,), F32)
            return 0
        lax.fori_loop(0, ns // 16, fill1, 0)
        pltpu.sync_copy(node_v, degacc.at[pl.ds(s * ns, ns)])
        plsc.subcore_barrier()

        def dbatch(b, _):
            base = s * ept + b * 128
            pltpu.sync_copy(dst_h.at[pl.ds(base, 128)], idx_v)
            pltpu.sync_copy(ew_h.at[pl.ds(base, 128)], w_v)
            pltpu.sync_copy(w_v, degacc.at[idx_v], add=True)
            return 0
        lax.fori_loop(0, nb1, dbatch, 0)
        plsc.subcore_barrier()

        pltpu.sync_copy(degacc.at[pl.ds(s * ns, ns)], node_v)

        def rsq(g, _):
            x = node_v[pl.ds(g * 16, 16)]
            i = lax.bitcast_convert_type(x, I32)
            i = jnp.int32(0x5F3759DF) - lax.shift_right_arithmetic(i, 1)
            y = lax.bitcast_convert_type(i, F32)
            xh = x * 0.5
            y = y * (1.5 - xh * y * y)
            y = y * (1.5 - xh * y * y)
            y = y * (1.5 - xh * y * y)
            node_v[pl.ds(g * 16, 16)] = y
            d2_v[pl.ds(g * 16, 16)] = y * y
            return 0
        lax.fori_loop(0, ns // 16, rsq, 0)
        pltpu.sync_copy(node_v, dinvsh.at[pl.ds(s * ns, ns)])

        @pl.when(c == 0)
        def _():
            pltpu.sync_copy(d2_v, d2_o.at[pl.ds(s * ns, ns)])

        plsc.subcore_barrier()
        pltpu.sync_copy(dinvsh, dinvt)

        def nbatch(b, _):
            base = c * (EP // 2) + s * ept2 + b * 64
            pltpu.sync_copy(src_h.at[pl.ds(base, 64)], src_v)
            pltpu.sync_copy(dst_h.at[pl.ds(base, 64)], dst_v)
            pltpu.sync_copy(ew_h.at[pl.ds(base, 64)], ew_v)

            def grp(g, _):
                ss = src_v[pl.ds(g * 16, 16)]
                dd = dst_v[pl.ds(g * 16, 16)]
                ww = ew_v[pl.ds(g * 16, 16)]
                a = plsc.load_gather(dinvt, [ss])
                bb = plsc.load_gather(dinvt, [dd])
                out_v[pl.ds(g * 16, 16)] = a * ww * bb
                return 0
            lax.fori_loop(0, 4, grp, 0)
            pltpu.sync_copy(out_v, norm_o.at[pl.ds(base, 64)])
            return 0
        lax.fori_loop(0, nb2, nbatch, 0)

    return k


def _sc_scatter(Npad, EP, Wh):
    rpt = Npad // 16
    zc = min(64, rpt)
    nz = rpt // zc
    wc = min(128, rpt)
    nw = rpt // wc
    ept = EP // 16
    nb = ept // 128
    ng = Wh // 16
    mesh = plsc.VectorSubcoreMesh(core_axis_name="c", subcore_axis_name="s")

    @partial(pl.kernel, mesh=mesh,
             compiler_params=pltpu.CompilerParams(needs_layout_passes=False, use_tc_tiling_on_sc=False),
             out_type=[jax.ShapeDtypeStruct((Npad, Wh), F32),
                       jax.ShapeDtypeStruct((Npad, Wh), F32)],
             scratch_types=[
                 pltpu.VMEM((128,), I32),
                 pltpu.VMEM((128,), I32),
                 pltpu.VMEM((128,), F32),
                 pltpu.VMEM((128, Wh), F32),
                 pltpu.VMEM((zc, Wh), F32),
                 pltpu.VMEM_SHARED((Npad, Wh), F32),
                 pltpu.SemaphoreType.DMA,
             ])
    def k(xa_h, xb_h, src_h, dst_h, norm_h, oa, ob,
          src_v, dst_v, norm_v, rows_v, zero_v, acc, sem):
        c = lax.axis_index("c")
        s = lax.axis_index("s")

        def zfill(i, _):
            r = i // ng
            j = i % ng
            zero_v[r, pl.ds(j * 16, 16)] = jnp.zeros((16,), F32)
            return 0
        lax.fori_loop(0, zc * ng, zfill, 0)

        def zcopy(i, _):
            pltpu.sync_copy(zero_v, acc.at[pl.ds(s * rpt + i * zc, zc)])
            return 0
        lax.fori_loop(0, nz, zcopy, 0)
        plsc.subcore_barrier()

        def batch(b, _):
            base = s * ept + b * 128
            pltpu.sync_copy(src_h.at[pl.ds(base, 128)], src_v)
            pltpu.sync_copy(dst_h.at[pl.ds(base, 128)], dst_v)
            pltpu.sync_copy(norm_h.at[pl.ds(base, 128)], norm_v)

            @pl.when(c == 0)
            def _():
                pltpu.async_copy(xa_h.at[src_v], rows_v, sem).wait()

            @pl.when(c == 1)
            def _():
                pltpu.async_copy(xb_h.at[src_v], rows_v, sem).wait()

            def scale(r, _):
                bc = plsc.load_gather(norm_v, [jnp.zeros((16,), I32) + r])
                for j in range(ng):
                    rows_v[r, pl.ds(j * 16, 16)] = rows_v[r, pl.ds(j * 16, 16)] * bc
                return 0
            lax.fori_loop(0, 128, scale, 0)
            pltpu.sync_copy(rows_v, acc.at[dst_v], add=True)
            return 0
        lax.fori_loop(0, nb, batch, 0)
        plsc.subcore_barrier()

        def wout(i, _):
            r0 = s * rpt + i * wc
            pltpu.sync_copy(acc.at[pl.ds(r0, wc)], rows_v.at[pl.ds(0, wc)])

            @pl.when(c == 0)
            def _():
                pltpu.sync_copy(rows_v.at[pl.ds(0, wc)], oa.at[pl.ds(r0, wc)])

            @pl.when(c == 1)
            def _():
                pltpu.sync_copy(rows_v.at[pl.ds(0, wc)], ob.at[pl.ds(r0, wc)])
            return 0
        lax.fori_loop(0, nw, wout, 0)

    return k


def _tc_layer(N, BN, Din):
    G = N // BN
    Dh = Din // 2

    def body(sa, sb, x, d2, W0, b0, W1a, W1b, pa, pb):
        ax = jnp.concatenate([sa[...], sb[...]], axis=1) + d2[...] * x[...]
        t = jnp.maximum(
            jnp.dot(ax, W0[...], preferred_element_type=F32) + b0[...], 0.0)
        pa[...] = jnp.dot(t, W1a[...], preferred_element_type=F32)
        pb[...] = jnp.dot(t, W1b[...], preferred_element_type=F32)

    return pl.pallas_call(
        body,
        grid=(G,),
        in_specs=[
            pl.BlockSpec((BN, Dh), lambda i: (i, 0)),
            pl.BlockSpec((BN, Dh), lambda i: (i, 0)),
            pl.BlockSpec((BN, Din), lambda i: (i, 0)),
            pl.BlockSpec((BN, 1), lambda i: (i, 0)),
            pl.BlockSpec((Din, _EMB), lambda i: (0, 0)),
            pl.BlockSpec((1, _EMB), lambda i: (0, 0)),
            pl.BlockSpec((_EMB, _EMB // 2), lambda i: (0, 0)),
            pl.BlockSpec((_EMB, _EMB // 2), lambda i: (0, 0)),
        ],
        out_specs=[
            pl.BlockSpec((BN, _EMB // 2), lambda i: (i, 0)),
            pl.BlockSpec((BN, _EMB // 2), lambda i: (i, 0)),
        ],
        out_shape=[
            jax.ShapeDtypeStruct((N, _EMB // 2), F32),
            jax.ShapeDtypeStruct((N, _EMB // 2), F32),
        ],
    )


def _tc_mean(N, BN):
    G = N // BN
    Dh = _EMB // 2

    def body(sa, sb, pa, pb, d2, b1, out):
        i = pl.program_id(0)
        e = jnp.concatenate([sa[...], sb[...]], axis=1) \
            + d2[...] * jnp.concatenate([pa[...], pb[...]], axis=1) + b1[...]
        e = jnp.maximum(e, 0.0)
        ss = jnp.sum(e * e, axis=1, keepdims=True)
        w = 1.0 / jnp.maximum(jnp.sqrt(ss), 1e-12)
        contrib = jnp.sum(w * e, axis=0, keepdims=True)

        @pl.when(i == 0)
        def _():
            out[...] = jnp.zeros_like(out)
        out[...] += contrib

    return pl.pallas_call(
        body,
        grid=(G,),
        in_specs=[
            pl.BlockSpec((BN, Dh), lambda i: (i, 0)),
            pl.BlockSpec((BN, Dh), lambda i: (i, 0)),
            pl.BlockSpec((BN, Dh), lambda i: (i, 0)),
            pl.BlockSpec((BN, Dh), lambda i: (i, 0)),
            pl.BlockSpec((BN, 1), lambda i: (i, 0)),
            pl.BlockSpec((1, _EMB), lambda i: (0, 0)),
        ],
        out_specs=pl.BlockSpec((1, _EMB), lambda i: (0, 0)),
        out_shape=jax.ShapeDtypeStruct((1, _EMB), F32),
    )


def _mish(x):
    return x * jnp.tanh(jax.nn.softplus(x))


def _tc_final(accn, accd, time2, act, Wt1, bt1, Wt2, bt2,
              Wf1, bf1, Wf2, bf2, Wf3, bf3):
    def body(an, ad, tm, ac, wt1, bt1_, wt2, bt2_, wf1, bf1_, wf2, bf2_,
             wf3, bf3_, out):
        hyb = an[...] * (1.0 / _N_NET) + ad[...] * (1.0 / _N_DAG)
        freqs = jnp.exp(lax.broadcasted_iota(I32, (1, 16), 1).astype(F32)
                        * jnp.float32(-np.log(10000.0) / 15.0))
        e = tm[...] * freqs
        pe = jnp.concatenate([jnp.sin(e), jnp.cos(e)], axis=1)
        temb = jnp.dot(_mish(jnp.dot(pe, wt1[...], preferred_element_type=F32)
                             + bt1_[...]),
                       wt2[...], preferred_element_type=F32) + bt2_[...]
        fi = jnp.concatenate([hyb, temb, ac[...]], axis=1)
        z = _mish(jnp.dot(fi, wf1[...], preferred_element_type=F32) + bf1_[...])
        z = jnp.dot(z, wf2[...], preferred_element_type=F32) + bf2_[...]
        out[...] = jnp.dot(z, wf3[...], preferred_element_type=F32) + bf3_[...]

    return pl.pallas_call(
        body,
        out_shape=jax.ShapeDtypeStruct((1, bf3.shape[-1]), F32),
    )(accn, accd, time2, act, Wt1, bt1, Wt2, bt2, Wf1, bf1, Wf2, bf2, Wf3, bf3)


def _pad1(x, n):
    pad = n - x.shape[0]
    return jnp.concatenate([x, jnp.zeros((pad,), x.dtype)])


def _gcn_stack(x, edge_index, edge_weight, W0, b0, W1, b1,
               N, Npad, EP, BN):
    Din = x.shape[1]
    Dh = Din // 2
    src = _pad1(edge_index[0], EP)
    dst = _pad1(edge_index[1], EP)
    ew = _pad1(edge_weight, EP)

    norm, d2 = _sc_norm(Npad, EP)(src, dst, ew)
    d2c = d2[:N, None]

    xa = x[:, :Dh]
    xb = x[:, Dh:]
    s0a, s0b = _sc_scatter(Npad, EP, Dh)(xa, xb, src, dst, norm)
    s0a, s0b = s0a[:N], s0b[:N]

    W1a, W1b = W1[:, :_EMB // 2], W1[:, _EMB // 2:]
    p1a, p1b = _tc_layer(N, BN, Din)(
        s0a, s0b, x, d2c, W0, b0[None, :], W1a, W1b)

    s1a, s1b = _sc_scatter(Npad, EP, _EMB // 2)(p1a, p1b, src, dst, norm)
    s1a, s1b = s1a[:N], s1b[:N]

    return _tc_mean(N, BN)(s1a, s1b, p1a, p1b, d2c, b1[None, :])


def kernel(action, time, net_feature, net_edge_index, net_edge_weight,
           dag_feature, dag_edge_index, dag_edge_weight, batch_size,
           Wn0, bn0, Wn1, bn1, Wd0, bd0, Wd1, bd1,
           Wt1, bt1, Wt2, bt2, Wf1, bf1, Wf2, bf2, Wf3, bf3):
    accn = _gcn_stack(net_feature, net_edge_index, net_edge_weight,
                      Wn0, bn0, Wn1, bn1, _N_NET, _NP_NET, _EP_NET, 1000)
    accd = _gcn_stack(dag_feature, dag_edge_index, dag_edge_weight,
                      Wd0, bd0, Wd1, bd1, _N_DAG, _NP_DAG, _EP_DAG, 1000)
    return _tc_final(accn, accd, time[:, None], action,
                     Wt1, bt1[None, :], Wt2, bt2[None, :],
                     Wf1, bf1[None, :], Wf2, bf2[None, :],
                     Wf3, bf3[None, :])


# speedup vs baseline: 7.9477x; 1.2274x over previous
import functools
from functools import partial

import numpy as np
import jax
import jax.numpy as jnp
from jax import lax
from jax.experimental import pallas as pl
from jax.experimental.pallas import tpu as pltpu
from jax.experimental.pallas import tpu_sc as plsc

F32 = jnp.float32
I32 = jnp.int32

_N_NET, _E_NET = 10000, 320000
_N_DAG, _E_DAG = 1000, 8000
_NP_NET, _EP_NET = 10240, 160 * 2048
_NP_DAG, _EP_DAG = 1024, 4 * 2048
_EMB = 256

_SC_PARAMS = pltpu.CompilerParams(
    needs_layout_passes=False, use_tc_tiling_on_sc=False)


def _sc_norm(Npad, EP):
    RW = EP // 128
    nbt = RW // 16
    fk = 8 if nbt % 8 == 0 else nbt
    nck = nbt // fk
    ns = Npad // 16
    hb = nbt // 2
    mesh = plsc.VectorSubcoreMesh(core_axis_name="c", subcore_axis_name="s")

    @partial(pl.kernel, mesh=mesh,
             compiler_params=_SC_PARAMS,
             out_type=[jax.ShapeDtypeStruct((RW, 128), F32),
                       jax.ShapeDtypeStruct((Npad,), F32)],
             scratch_types=[
                 pltpu.VMEM((nbt, 2, 128), I32),
                 pltpu.VMEM((nbt, 128), F32),
                 pltpu.VMEM((hb, 128), F32),
                 pltpu.VMEM((ns,), F32),
                 pltpu.VMEM((ns,), F32),
                 pltpu.VMEM((Npad,), F32),
                 pltpu.VMEM_SHARED((Npad,), F32),
                 pltpu.VMEM_SHARED((Npad,), F32),
                 pltpu.SemaphoreType.DMA,
             ])
    def k(sd_h, ew_h, norm_o, d2_o,
          sdb, ewb, normout, node_v, d2_v, dinvt, degacc, dinvsh, dsem):
        c = lax.axis_index("c")
        s = lax.axis_index("s")

        pltpu.sync_copy(sd_h.at[pl.ds(s * nbt, nbt)], sdb)
        pltpu.sync_copy(ew_h.at[pl.ds(s * nbt, nbt)], ewb)

        def fill1(i, _):
            node_v[pl.ds(i * 16, 16)] = jnp.ones((16,), F32)
            return 0
        lax.fori_loop(0, ns // 16, fill1, 0)
        pltpu.sync_copy(node_v, degacc.at[pl.ds(s * ns, ns)])
        plsc.subcore_barrier()

        def dchunk(q, _):
            for j in range(fk):
                b = q * fk + j
                pltpu.async_copy(ewb.at[b], degacc.at[sdb.at[b, 1]],
                                 dsem, add=True)
            for j in range(fk):
                pltpu.make_async_copy(
                    ewb.at[0], degacc.at[sdb.at[0, 1]], dsem).wait()
            return 0
        lax.fori_loop(0, nck, dchunk, 0)
        plsc.subcore_barrier()

        pltpu.sync_copy(degacc.at[pl.ds(s * ns, ns)], node_v)

        def rsq(g, _):
            x = node_v[pl.ds(g * 16, 16)]
            i = lax.bitcast_convert_type(x, I32)
            i = jnp.int32(0x5F3759DF) - lax.shift_right_arithmetic(i, 1)
            y = lax.bitcast_convert_type(i, F32)
            xh = x * 0.5
            y = y * (1.5 - xh * y * y)
            y = y * (1.5 - xh * y * y)
            y = y * (1.5 - xh * y * y)
            node_v[pl.ds(g * 16, 16)] = y
            d2_v[pl.ds(g * 16, 16)] = y * y
            return 0
        lax.fori_loop(0, ns // 16, rsq, 0)
        pltpu.sync_copy(node_v, dinvsh.at[pl.ds(s * ns, ns)])

        @pl.when(c == 0)
        def _():
            pltpu.sync_copy(d2_v, d2_o.at[pl.ds(s * ns, ns)])

        plsc.subcore_barrier()
        pltpu.sync_copy(dinvsh, dinvt)

        def nrow(r, _):
            b = c * hb + r

            def grp(g, _):
                ss = sdb[b, 0, pl.ds(g * 16, 16)]
                dd = sdb[b, 1, pl.ds(g * 16, 16)]
                ww = ewb[b, pl.ds(g * 16, 16)]
                a = plsc.load_gather(dinvt, [ss])
                bb = plsc.load_gather(dinvt, [dd])
                normout[r, pl.ds(g * 16, 16)] = a * ww * bb
                return 0
            lax.fori_loop(0, 8, grp, 0)
            return 0
        lax.fori_loop(0, hb, nrow, 0)
        pltpu.sync_copy(normout, norm_o.at[pl.ds(s * nbt + c * hb, hb)])

    return k


def _sc_scatter(Npad, EP, Wh):
    RW = EP // 128
    nbt = RW // 16
    rpt = Npad // 16
    zc = min(64, rpt)
    nz = rpt // zc
    wc = min(128, rpt)
    nw = rpt // wc
    ng = Wh // 16
    mesh = plsc.VectorSubcoreMesh(core_axis_name="c", subcore_axis_name="s")

    @partial(pl.kernel, mesh=mesh,
             compiler_params=_SC_PARAMS,
             out_type=[jax.ShapeDtypeStruct((Npad, Wh), F32),
                       jax.ShapeDtypeStruct((Npad, Wh), F32)],
             scratch_types=[
                 pltpu.VMEM((nbt, 2, 128), I32),
                 pltpu.VMEM((nbt * 128,), F32),
                 pltpu.VMEM((2, 128, Wh), F32),
                 pltpu.VMEM((zc, Wh), F32),
                 pltpu.VMEM_SHARED((Npad, Wh), F32),
                 pltpu.SemaphoreType.DMA,
                 pltpu.SemaphoreType.DMA,
                 pltpu.SemaphoreType.DMA,
                 pltpu.SemaphoreType.DMA,
             ])
    def k(xa_h, xb_h, sd_h, norm_h, oa, ob,
          sdb, normb, rows, zero_v, acc, gs0, gs1, ss0, ss1):
        c = lax.axis_index("c")
        s = lax.axis_index("s")
        gsem = (gs0, gs1)
        ssem = (ss0, ss1)

        pltpu.sync_copy(sd_h.at[pl.ds(s * nbt, nbt)], sdb)
        pltpu.sync_copy(norm_h.at[pl.ds(s * nbt * 128, nbt * 128)], normb)

        def zfill(i, _):
            r = i // ng
            j = i % ng
            zero_v[r, pl.ds(j * 16, 16)] = jnp.zeros((16,), F32)
            return 0
        lax.fori_loop(0, zc * ng, zfill, 0)

        def zcopy(i, _):
            pltpu.sync_copy(zero_v, acc.at[pl.ds(s * rpt + i * zc, zc)])
            return 0
        lax.fori_loop(0, nz, zcopy, 0)
        plsc.subcore_barrier()

        def start_g(b, j):
            @pl.when(c == 0)
            def _():
                pltpu.async_copy(xa_h.at[sdb.at[b, 0]], rows.at[j], gsem[j])

            @pl.when(c == 1)
            def _():
                pltpu.async_copy(xb_h.at[sdb.at[b, 0]], rows.at[j], gsem[j])

        def wait_g(j):
            pltpu.make_async_copy(
                xa_h.at[sdb.at[0, 0]], rows.at[j], gsem[j]).wait()

        def scale(b, j):
            def srow(r, _):
                bc = plsc.load_gather(normb,
                                      [jnp.zeros((16,), I32) + b * 128 + r])
                for g in range(ng):
                    rows[j, r, pl.ds(g * 16, 16)] = (
                        rows[j, r, pl.ds(g * 16, 16)] * bc)
                return 0
            lax.fori_loop(0, 128, srow, 0)

        def start_s(b, j):
            pltpu.async_copy(rows.at[j], acc.at[sdb.at[b, 1]], ssem[j],
                             add=True)

        def wait_s(j):
            pltpu.make_async_copy(
                rows.at[j], acc.at[sdb.at[0, 1]], ssem[j]).wait()

        nloop = nbt // 2
        start_g(0, 0)

        def body(bb, _):
            b0 = 2 * bb
            b1 = b0 + 1
            start_g(b1, 1)
            wait_g(0)
            scale(b0, 0)
            start_s(b0, 0)
            wait_g(1)
            scale(b1, 1)
            start_s(b1, 1)
            wait_s(0)

            @pl.when(bb + 1 < nloop)
            def _():
                start_g(b0 + 2, 0)
            wait_s(1)
            return 0
        lax.fori_loop(0, nloop, body, 0)
        plsc.subcore_barrier()

        def wout(i, _):
            r0 = s * rpt + i * wc
            pltpu.sync_copy(acc.at[pl.ds(r0, wc)], rows.at[0, pl.ds(0, wc)])

            @pl.when(c == 0)
            def _():
                pltpu.sync_copy(rows.at[0, pl.ds(0, wc)], oa.at[pl.ds(r0, wc)])

            @pl.when(c == 1)
            def _():
                pltpu.sync_copy(rows.at[0, pl.ds(0, wc)], ob.at[pl.ds(r0, wc)])
            return 0
        lax.fori_loop(0, nw, wout, 0)

    return k


def _tc_layer(N, BN, Din):
    G = N // BN
    Dh = Din // 2

    def body(sa, sb, x, d2, W0, b0, W1a, W1b, W1c, W1d, pa, pb, pc, pd):
        ax = jnp.concatenate([sa[...], sb[...]], axis=1) + d2[...] * x[...]
        t = jnp.maximum(
            jnp.dot(ax, W0[...], preferred_element_type=F32) + b0[...], 0.0)
        pa[...] = jnp.dot(t, W1a[...], preferred_element_type=F32)
        pb[...] = jnp.dot(t, W1b[...], preferred_element_type=F32)
        pc[...] = jnp.dot(t, W1c[...], preferred_element_type=F32)
        pd[...] = jnp.dot(t, W1d[...], preferred_element_type=F32)

    Q = _EMB // 4
    return pl.pallas_call(
        body,
        grid=(G,),
        in_specs=[
            pl.BlockSpec((BN, Dh), lambda i: (i, 0)),
            pl.BlockSpec((BN, Dh), lambda i: (i, 0)),
            pl.BlockSpec((BN, Din), lambda i: (i, 0)),
            pl.BlockSpec((BN, 1), lambda i: (i, 0)),
            pl.BlockSpec((Din, _EMB), lambda i: (0, 0)),
            pl.BlockSpec((1, _EMB), lambda i: (0, 0)),
        ] + [pl.BlockSpec((_EMB, Q), lambda i: (0, 0))] * 4,
        out_specs=[pl.BlockSpec((BN, Q), lambda i: (i, 0))] * 4,
        out_shape=[jax.ShapeDtypeStruct((N, Q), F32)] * 4,
    )


def _tc_mean(N, BN):
    G = N // BN
    Q = _EMB // 4

    def body(s0, s1, s2, s3, p0, p1, p2, p3, d2, b1, out):
        i = pl.program_id(0)
        e = jnp.concatenate([s0[...], s1[...], s2[...], s3[...]], axis=1) \
            + d2[...] * jnp.concatenate(
                [p0[...], p1[...], p2[...], p3[...]], axis=1) + b1[...]
        e = jnp.maximum(e, 0.0)
        ss = jnp.sum(e * e, axis=1, keepdims=True)
        w = 1.0 / jnp.maximum(jnp.sqrt(ss), 1e-12)
        contrib = jnp.sum(w * e, axis=0, keepdims=True)

        @pl.when(i == 0)
        def _():
            out[...] = jnp.zeros_like(out)
        out[...] += contrib

    return pl.pallas_call(
        body,
        grid=(G,),
        in_specs=[pl.BlockSpec((BN, Q), lambda i: (i, 0))] * 8 + [
            pl.BlockSpec((BN, 1), lambda i: (i, 0)),
            pl.BlockSpec((1, _EMB), lambda i: (0, 0)),
        ],
        out_specs=pl.BlockSpec((1, _EMB), lambda i: (0, 0)),
        out_shape=jax.ShapeDtypeStruct((1, _EMB), F32),
    )


def _mish(x):
    return x * jnp.tanh(jax.nn.softplus(x))


def _tc_final(accn, accd, time2, act, Wt1, bt1, Wt2, bt2,
              Wf1, bf1, Wf2, bf2, Wf3, bf3):
    def body(an, ad, tm, ac, wt1, bt1_, wt2, bt2_, wf1, bf1_, wf2, bf2_,
             wf3, bf3_, out):
        hyb = an[...] * (1.0 / _N_NET) + ad[...] * (1.0 / _N_DAG)
        freqs = jnp.exp(lax.broadcasted_iota(I32, (1, 16), 1).astype(F32)
                        * jnp.float32(-np.log(10000.0) / 15.0))
        e = tm[...] * freqs
        pe = jnp.concatenate([jnp.sin(e), jnp.cos(e)], axis=1)
        temb = jnp.dot(_mish(jnp.dot(pe, wt1[...], preferred_element_type=F32)
                             + bt1_[...]),
                       wt2[...], preferred_element_type=F32) + bt2_[...]
        fi = jnp.concatenate([hyb, temb, ac[...]], axis=1)
        z = _mish(jnp.dot(fi, wf1[...], preferred_element_type=F32) + bf1_[...])
        z = jnp.dot(z, wf2[...], preferred_element_type=F32) + bf2_[...]
        out[...] = jnp.dot(z, wf3[...], preferred_element_type=F32) + bf3_[...]

    return pl.pallas_call(
        body,
        out_shape=jax.ShapeDtypeStruct((1, bf3.shape[-1]), F32),
    )(accn, accd, time2, act, Wt1, bt1, Wt2, bt2, Wf1, bf1, Wf2, bf2, Wf3, bf3)


def _pad1(x, n):
    pad = n - x.shape[0]
    return jnp.concatenate([x, jnp.zeros((pad,), x.dtype)])


def _gcn_stack(x, edge_index, edge_weight, W0, b0, W1, b1,
               N, Npad, EP, BN):
    Din = x.shape[1]
    Dh = Din // 2
    src = _pad1(edge_index[0], EP).reshape(EP // 128, 128)
    dst = _pad1(edge_index[1], EP).reshape(EP // 128, 128)
    ew = _pad1(edge_weight, EP).reshape(EP // 128, 128)
    sd = jnp.stack([src, dst], axis=1)

    norm2, d2 = _sc_norm(Npad, EP)(sd, ew)
    norm = norm2.reshape(EP)
    d2c = d2[:N, None]

    xa = x[:, :Dh]
    xb = x[:, Dh:]
    s0a, s0b = _sc_scatter(Npad, EP, Dh)(xa, xb, sd, norm)
    s0a, s0b = s0a[:N], s0b[:N]

    Q = _EMB // 4
    W1q = [W1[:, i * Q:(i + 1) * Q] for i in range(4)]
    p1 = _tc_layer(N, BN, Din)(
        s0a, s0b, x, d2c, W0, b0[None, :], *W1q)

    scat_q = _sc_scatter(Npad, EP, Q)
    s1a, s1b = scat_q(p1[0], p1[1], sd, norm)
    s1c, s1d = scat_q(p1[2], p1[3], sd, norm)
    s1 = [v[:N] for v in (s1a, s1b, s1c, s1d)]

    return _tc_mean(N, BN)(*s1, *p1, d2c, b1[None, :])


def kernel(action, time, net_feature, net_edge_index, net_edge_weight,
           dag_feature, dag_edge_index, dag_edge_weight, batch_size,
           Wn0, bn0, Wn1, bn1, Wd0, bd0, Wd1, bd1,
           Wt1, bt1, Wt2, bt2, Wf1, bf1, Wf2, bf2, Wf3, bf3):
    accn = _gcn_stack(net_feature, net_edge_index, net_edge_weight,
                      Wn0, bn0, Wn1, bn1, _N_NET, _NP_NET, _EP_NET, 1000)
    accd = _gcn_stack(dag_feature, dag_edge_index, dag_edge_weight,
                      Wd0, bd0, Wd1, bd1, _N_DAG, _NP_DAG, _EP_DAG, 1000)
    return _tc_final(accn, accd, time[:, None], action,
                     Wt1, bt1[None, :], Wt2, bt2[None, :],
                     Wf1, bf1[None, :], Wf2, bf2[None, :],
                     Wf3, bf3[None, :])


# speedup vs baseline: 8.5411x; 1.0747x over previous
import functools
from functools import partial

import numpy as np
import jax
import jax.numpy as jnp
from jax import lax
from jax.experimental import pallas as pl
from jax.experimental.pallas import tpu as pltpu
from jax.experimental.pallas import tpu_sc as plsc

F32 = jnp.float32
I32 = jnp.int32

_N_NET, _E_NET = 10000, 320000
_N_DAG, _E_DAG = 1000, 8000
_NP_NET, _EP_NET = 10240, 160 * 2048
_NP_DAG, _EP_DAG = 1024, 4 * 2048
_EMB = 256

_SC_PARAMS = pltpu.CompilerParams(
    needs_layout_passes=False, use_tc_tiling_on_sc=False)


def _sc_norm(Npad, EP):
    RW = EP // 128
    nbt = RW // 16
    fk = 8 if nbt % 8 == 0 else nbt
    nck = nbt // fk
    ns = Npad // 16
    hb = nbt // 2
    mesh = plsc.VectorSubcoreMesh(core_axis_name="c", subcore_axis_name="s")

    @partial(pl.kernel, mesh=mesh,
             compiler_params=_SC_PARAMS,
             out_type=[jax.ShapeDtypeStruct((RW, 128), F32),
                       jax.ShapeDtypeStruct((Npad,), F32)],
             scratch_types=[
                 pltpu.VMEM((nbt, 2, 128), I32),
                 pltpu.VMEM((nbt, 128), F32),
                 pltpu.VMEM((hb, 128), F32),
                 pltpu.VMEM((ns,), F32),
                 pltpu.VMEM((ns,), F32),
                 pltpu.VMEM((Npad,), F32),
                 pltpu.VMEM_SHARED((Npad,), F32),
                 pltpu.VMEM_SHARED((Npad,), F32),
                 pltpu.SemaphoreType.DMA,
             ])
    def k(sd_h, ew_h, norm_o, d2_o,
          sdb, ewb, normout, node_v, d2_v, dinvt, degacc, dinvsh, dsem):
        c = lax.axis_index("c")
        s = lax.axis_index("s")

        pltpu.sync_copy(sd_h.at[pl.ds(s * nbt, nbt)], sdb)
        pltpu.sync_copy(ew_h.at[pl.ds(s * nbt, nbt)], ewb)

        def fill1(i, _):
            node_v[pl.ds(i * 16, 16)] = jnp.ones((16,), F32)
            return 0
        lax.fori_loop(0, ns // 16, fill1, 0)
        pltpu.sync_copy(node_v, degacc.at[pl.ds(s * ns, ns)])
        plsc.subcore_barrier()

        def dchunk(q, _):
            for j in range(fk):
                b = q * fk + j
                pltpu.async_copy(ewb.at[b], degacc.at[sdb.at[b, 1]],
                                 dsem, add=True)
            for j in range(fk):
                pltpu.make_async_copy(
                    ewb.at[0], degacc.at[sdb.at[0, 1]], dsem).wait()
            return 0
        lax.fori_loop(0, nck, dchunk, 0)
        plsc.subcore_barrier()

        pltpu.sync_copy(degacc.at[pl.ds(s * ns, ns)], node_v)

        def rsq(g, _):
            x = node_v[pl.ds(g * 16, 16)]
            i = lax.bitcast_convert_type(x, I32)
            i = jnp.int32(0x5F3759DF) - lax.shift_right_arithmetic(i, 1)
            y = lax.bitcast_convert_type(i, F32)
            xh = x * 0.5
            y = y * (1.5 - xh * y * y)
            y = y * (1.5 - xh * y * y)
            y = y * (1.5 - xh * y * y)
            node_v[pl.ds(g * 16, 16)] = y
            d2_v[pl.ds(g * 16, 16)] = y * y
            return 0
        lax.fori_loop(0, ns // 16, rsq, 0)
        pltpu.sync_copy(node_v, dinvsh.at[pl.ds(s * ns, ns)])

        @pl.when(c == 0)
        def _():
            pltpu.sync_copy(d2_v, d2_o.at[pl.ds(s * ns, ns)])

        plsc.subcore_barrier()
        pltpu.sync_copy(dinvsh, dinvt)

        def nrow(r, _):
            b = c * hb + r

            def grp(g, _):
                ss = sdb[b, 0, pl.ds(g * 16, 16)]
                dd = sdb[b, 1, pl.ds(g * 16, 16)]
                ww = ewb[b, pl.ds(g * 16, 16)]
                a = plsc.load_gather(dinvt, [ss])
                bb = plsc.load_gather(dinvt, [dd])
                normout[r, pl.ds(g * 16, 16)] = a * ww * bb
                return 0
            lax.fori_loop(0, 8, grp, 0)
            return 0
        lax.fori_loop(0, hb, nrow, 0)
        pltpu.sync_copy(normout, norm_o.at[pl.ds(s * nbt + c * hb, hb)])

    return k


def _sc_scatter(Npad, EP, Wh):
    RW = EP // 128
    nbt = RW // 16
    rpt = Npad // 16
    zc = min(16, rpt)
    nz = rpt // zc
    wc = min(128, rpt)
    nw = rpt // wc
    ng = Wh // 16
    mesh = plsc.VectorSubcoreMesh(core_axis_name="c", subcore_axis_name="s")

    @partial(pl.kernel, mesh=mesh,
             compiler_params=_SC_PARAMS,
             out_type=[jax.ShapeDtypeStruct((Npad, Wh), F32),
                       jax.ShapeDtypeStruct((Npad, Wh), F32)],
             scratch_types=[
                 pltpu.VMEM((nbt, 2, 128), I32),
                 pltpu.VMEM((512,), F32),
                 pltpu.VMEM((2, 256, Wh), F32),
                 pltpu.VMEM((zc, Wh), F32),
                 pltpu.VMEM_SHARED((Npad, Wh), F32),
                 pltpu.SemaphoreType.DMA,
                 pltpu.SemaphoreType.DMA,
                 pltpu.SemaphoreType.DMA,
                 pltpu.SemaphoreType.DMA,
                 pltpu.SemaphoreType.DMA,
                 pltpu.SemaphoreType.DMA,
             ])
    def k(xa_h, xb_h, sd_h, norm_h, oa, ob,
          sdb, normb, rows, zero_v, acc, gs0, gs1, ss0, ss1, ns0, ns1):
        c = lax.axis_index("c")
        s = lax.axis_index("s")
        gsem = (gs0, gs1)
        ssem = (ss0, ss1)
        nsem = (ns0, ns1)

        pltpu.sync_copy(sd_h.at[pl.ds(s * nbt, nbt)], sdb)

        def zfill(i, _):
            r = i // ng
            j = i % ng
            zero_v[r, pl.ds(j * 16, 16)] = jnp.zeros((16,), F32)
            return 0
        lax.fori_loop(0, zc * ng, zfill, 0)

        def zcopy(i, _):
            pltpu.sync_copy(zero_v, acc.at[pl.ds(s * rpt + i * zc, zc)])
            return 0
        lax.fori_loop(0, nz, zcopy, 0)
        plsc.subcore_barrier()

        def start_g(sb, j):
            @pl.when(c == 0)
            def _():
                pltpu.async_copy(xa_h.at[sdb.at[2 * sb, 0]],
                                 rows.at[j, pl.ds(0, 128)], gsem[j])
                pltpu.async_copy(xa_h.at[sdb.at[2 * sb + 1, 0]],
                                 rows.at[j, pl.ds(128, 128)], gsem[j])

            @pl.when(c == 1)
            def _():
                pltpu.async_copy(xb_h.at[sdb.at[2 * sb, 0]],
                                 rows.at[j, pl.ds(0, 128)], gsem[j])
                pltpu.async_copy(xb_h.at[sdb.at[2 * sb + 1, 0]],
                                 rows.at[j, pl.ds(128, 128)], gsem[j])

        def wait_g(j):
            for _ in range(2):
                pltpu.make_async_copy(
                    xa_h.at[sdb.at[0, 0]],
                    rows.at[j, pl.ds(0, 128)], gsem[j]).wait()

        def start_n(sb, j):
            pltpu.async_copy(
                norm_h.at[pl.ds(s * nbt * 128 + sb * 256, 256)],
                normb.at[pl.ds(j * 256, 256)], nsem[j])

        def wait_n(j):
            pltpu.make_async_copy(
                norm_h.at[pl.ds(0, 256)],
                normb.at[pl.ds(j * 256, 256)], nsem[j]).wait()

        def scale(sb, j):
            def srow(rr, _):
                for u in range(2):
                    r = rr * 2 + u
                    bc = plsc.load_gather(
                        normb, [jnp.zeros((16,), I32) + j * 256 + r])
                    for g in range(ng):
                        rows[j, r, pl.ds(g * 16, 16)] = (
                            rows[j, r, pl.ds(g * 16, 16)] * bc)
                return 0
            lax.fori_loop(0, 128, srow, 0)

        def start_s(sb, j):
            pltpu.async_copy(rows.at[j, pl.ds(0, 128)],
                             acc.at[sdb.at[2 * sb, 1]], ssem[j], add=True)
            pltpu.async_copy(rows.at[j, pl.ds(128, 128)],
                             acc.at[sdb.at[2 * sb + 1, 1]], ssem[j], add=True)

        def wait_s(j):
            for _ in range(2):
                pltpu.make_async_copy(
                    rows.at[j, pl.ds(0, 128)],
                    acc.at[sdb.at[0, 1]], ssem[j]).wait()

        nloop = nbt // 4
        start_g(0, 0)
        start_n(0, 0)

        def body(bb, _):
            s0 = 2 * bb
            s1 = s0 + 1
            start_g(s1, 1)
            start_n(s1, 1)
            wait_g(0)
            wait_n(0)
            scale(s0, 0)
            start_s(s0, 0)
            wait_g(1)
            wait_n(1)
            scale(s1, 1)
            start_s(s1, 1)
            wait_s(0)

            @pl.when(bb + 1 < nloop)
            def _():
                start_g(s0 + 2, 0)
                start_n(s0 + 2, 0)
            wait_s(1)
            return 0
        lax.fori_loop(0, nloop, body, 0)
        plsc.subcore_barrier()

        def wout(i, _):
            r0 = s * rpt + i * wc
            pltpu.sync_copy(acc.at[pl.ds(r0, wc)], rows.at[0, pl.ds(0, wc)])

            @pl.when(c == 0)
            def _():
                pltpu.sync_copy(rows.at[0, pl.ds(0, wc)], oa.at[pl.ds(r0, wc)])

            @pl.when(c == 1)
            def _():
                pltpu.sync_copy(rows.at[0, pl.ds(0, wc)], ob.at[pl.ds(r0, wc)])
            return 0
        lax.fori_loop(0, nw, wout, 0)

    return k


def _tc_layer(N, BN, Din):
    G = N // BN
    Dh = Din // 2

    def body(sa, sb, x, d2, W0, b0, W1a, W1b, W1c, W1d, pa, pb, pc, pd):
        ax = jnp.concatenate([sa[...], sb[...]], axis=1) + d2[...] * x[...]
        t = jnp.maximum(
            jnp.dot(ax, W0[...], preferred_element_type=F32) + b0[...], 0.0)
        pa[...] = jnp.dot(t, W1a[...], preferred_element_type=F32)
        pb[...] = jnp.dot(t, W1b[...], preferred_element_type=F32)
        pc[...] = jnp.dot(t, W1c[...], preferred_element_type=F32)
        pd[...] = jnp.dot(t, W1d[...], preferred_element_type=F32)

    Q = _EMB // 4
    return pl.pallas_call(
        body,
        grid=(G,),
        in_specs=[
            pl.BlockSpec((BN, Dh), lambda i: (i, 0)),
            pl.BlockSpec((BN, Dh), lambda i: (i, 0)),
            pl.BlockSpec((BN, Din), lambda i: (i, 0)),
            pl.BlockSpec((BN, 1), lambda i: (i, 0)),
            pl.BlockSpec((Din, _EMB), lambda i: (0, 0)),
            pl.BlockSpec((1, _EMB), lambda i: (0, 0)),
        ] + [pl.BlockSpec((_EMB, Q), lambda i: (0, 0))] * 4,
        out_specs=[pl.BlockSpec((BN, Q), lambda i: (i, 0))] * 4,
        out_shape=[jax.ShapeDtypeStruct((N, Q), F32)] * 4,
    )


def _tc_mean(N, BN):
    G = N // BN
    Q = _EMB // 4

    def body(s0, s1, s2, s3, p0, p1, p2, p3, d2, b1, out):
        i = pl.program_id(0)
        e = jnp.concatenate([s0[...], s1[...], s2[...], s3[...]], axis=1) \
            + d2[...] * jnp.concatenate(
                [p0[...], p1[...], p2[...], p3[...]], axis=1) + b1[...]
        e = jnp.maximum(e, 0.0)
        ss = jnp.sum(e * e, axis=1, keepdims=True)
        w = 1.0 / jnp.maximum(jnp.sqrt(ss), 1e-12)
        contrib = jnp.sum(w * e, axis=0, keepdims=True)

        @pl.when(i == 0)
        def _():
            out[...] = jnp.zeros_like(out)
        out[...] += contrib

    return pl.pallas_call(
        body,
        grid=(G,),
        in_specs=[pl.BlockSpec((BN, Q), lambda i: (i, 0))] * 8 + [
            pl.BlockSpec((BN, 1), lambda i: (i, 0)),
            pl.BlockSpec((1, _EMB), lambda i: (0, 0)),
        ],
        out_specs=pl.BlockSpec((1, _EMB), lambda i: (0, 0)),
        out_shape=jax.ShapeDtypeStruct((1, _EMB), F32),
    )


def _mish(x):
    return x * jnp.tanh(jax.nn.softplus(x))


def _tc_final(accn, accd, time2, act, Wt1, bt1, Wt2, bt2,
              Wf1, bf1, Wf2, bf2, Wf3, bf3):
    def body(an, ad, tm, ac, wt1, bt1_, wt2, bt2_, wf1, bf1_, wf2, bf2_,
             wf3, bf3_, out):
        hyb = an[...] * (1.0 / _N_NET) + ad[...] * (1.0 / _N_DAG)
        freqs = jnp.exp(lax.broadcasted_iota(I32, (1, 16), 1).astype(F32)
                        * jnp.float32(-np.log(10000.0) / 15.0))
        e = tm[...] * freqs
        pe = jnp.concatenate([jnp.sin(e), jnp.cos(e)], axis=1)
        temb = jnp.dot(_mish(jnp.dot(pe, wt1[...], preferred_element_type=F32)
                             + bt1_[...]),
                       wt2[...], preferred_element_type=F32) + bt2_[...]
        fi = jnp.concatenate([hyb, temb, ac[...]], axis=1)
        z = _mish(jnp.dot(fi, wf1[...], preferred_element_type=F32) + bf1_[...])
        z = jnp.dot(z, wf2[...], preferred_element_type=F32) + bf2_[...]
        out[...] = jnp.dot(z, wf3[...], preferred_element_type=F32) + bf3_[...]

    return pl.pallas_call(
        body,
        out_shape=jax.ShapeDtypeStruct((1, bf3.shape[-1]), F32),
    )(accn, accd, time2, act, Wt1, bt1, Wt2, bt2, Wf1, bf1, Wf2, bf2, Wf3, bf3)


def _pad1(x, n):
    pad = n - x.shape[0]
    return jnp.concatenate([x, jnp.zeros((pad,), x.dtype)])


def _gcn_stack(x, edge_index, edge_weight, W0, b0, W1, b1,
               N, Npad, EP, BN):
    Din = x.shape[1]
    Dh = Din // 2
    src = _pad1(edge_index[0], EP).reshape(EP // 128, 128)
    dst = _pad1(edge_index[1], EP).reshape(EP // 128, 128)
    ew = _pad1(edge_weight, EP).reshape(EP // 128, 128)
    sd = jnp.stack([src, dst], axis=1)

    norm2, d2 = _sc_norm(Npad, EP)(sd, ew)
    norm = norm2.reshape(EP)
    d2c = d2[:N, None]

    xa = x[:, :Dh]
    xb = x[:, Dh:]
    s0a, s0b = _sc_scatter(Npad, EP, Dh)(xa, xb, sd, norm)
    s0a, s0b = s0a[:N], s0b[:N]

    Q = _EMB // 4
    W1q = [W1[:, i * Q:(i + 1) * Q] for i in range(4)]
    p1 = _tc_layer(N, BN, Din)(
        s0a, s0b, x, d2c, W0, b0[None, :], *W1q)

    scat_q = _sc_scatter(Npad, EP, Q)
    s1a, s1b = scat_q(p1[0], p1[1], sd, norm)
    s1c, s1d = scat_q(p1[2], p1[3], sd, norm)
    s1 = [v[:N] for v in (s1a, s1b, s1c, s1d)]

    return _tc_mean(N, BN)(*s1, *p1, d2c, b1[None, :])


def kernel(action, time, net_feature, net_edge_index, net_edge_weight,
           dag_feature, dag_edge_index, dag_edge_weight, batch_size,
           Wn0, bn0, Wn1, bn1, Wd0, bd0, Wd1, bd1,
           Wt1, bt1, Wt2, bt2, Wf1, bf1, Wf2, bf2, Wf3, bf3):
    accn = _gcn_stack(net_feature, net_edge_index, net_edge_weight,
                      Wn0, bn0, Wn1, bn1, _N_NET, _NP_NET, _EP_NET, 1000)
    accd = _gcn_stack(dag_feature, dag_edge_index, dag_edge_weight,
                      Wd0, bd0, Wd1, bd1, _N_DAG, _NP_DAG, _EP_DAG, 1000)
    return _tc_final(accn, accd, time[:, None], action,
                     Wt1, bt1[None, :], Wt2, bt2[None, :],
                     Wf1, bf1[None, :], Wf2, bf2[None, :],
                     Wf3, bf3[None, :])


# speedup vs baseline: 9.7578x; 1.1425x over previous
import functools
from functools import partial

import numpy as np
import jax
import jax.numpy as jnp
from jax import lax
from jax.experimental import pallas as pl
from jax.experimental.pallas import tpu as pltpu
from jax.experimental.pallas import tpu_sc as plsc

F32 = jnp.float32
I32 = jnp.int32

_N_NET, _E_NET = 10000, 320000
_N_DAG, _E_DAG = 1000, 8000
_NP_NET, _EP_NET = 10240, 160 * 2048
_NP_DAG, _EP_DAG = 1024, 4 * 2048
_EMB = 256

_SC_PARAMS = pltpu.CompilerParams(
    needs_layout_passes=False, use_tc_tiling_on_sc=False)


def _sc_norm(Npad, EP):
    RW = EP // 128
    nbt = RW // 16
    fk = 8 if nbt % 8 == 0 else nbt
    nck = nbt // fk
    ns = Npad // 16
    hb = nbt // 2
    mesh = plsc.VectorSubcoreMesh(core_axis_name="c", subcore_axis_name="s")

    @partial(pl.kernel, mesh=mesh,
             compiler_params=_SC_PARAMS,
             out_type=[jax.ShapeDtypeStruct((RW, 128), F32),
                       jax.ShapeDtypeStruct((Npad,), F32)],
             scratch_types=[
                 pltpu.VMEM((nbt, 2, 128), I32),
                 pltpu.VMEM((nbt, 128), F32),
                 pltpu.VMEM((hb, 128), F32),
                 pltpu.VMEM((ns,), F32),
                 pltpu.VMEM((ns,), F32),
                 pltpu.VMEM((Npad,), F32),
                 pltpu.VMEM_SHARED((Npad,), F32),
                 pltpu.VMEM_SHARED((Npad,), F32),
                 pltpu.SemaphoreType.DMA,
             ])
    def k(sd_h, ew_h, norm_o, d2_o,
          sdb, ewb, normout, node_v, d2_v, dinvt, degacc, dinvsh, dsem):
        c = lax.axis_index("c")
        s = lax.axis_index("s")

        pltpu.sync_copy(sd_h.at[pl.ds(s * nbt, nbt)], sdb)
        pltpu.sync_copy(ew_h.at[pl.ds(s * nbt, nbt)], ewb)

        def fill1(i, _):
            node_v[pl.ds(i * 16, 16)] = jnp.ones((16,), F32)
            return 0
        lax.fori_loop(0, ns // 16, fill1, 0)
        pltpu.sync_copy(node_v, degacc.at[pl.ds(s * ns, ns)])
        plsc.subcore_barrier()

        def dchunk(q, _):
            for j in range(fk):
                b = q * fk + j
                pltpu.async_copy(ewb.at[b], degacc.at[sdb.at[b, 1]],
                                 dsem, add=True)
            for j in range(fk):
                pltpu.make_async_copy(
                    ewb.at[0], degacc.at[sdb.at[0, 1]], dsem).wait()
            return 0
        lax.fori_loop(0, nck, dchunk, 0)
        plsc.subcore_barrier()

        pltpu.sync_copy(degacc.at[pl.ds(s * ns, ns)], node_v)

        def rsq(g, _):
            x = node_v[pl.ds(g * 16, 16)]
            i = lax.bitcast_convert_type(x, I32)
            i = jnp.int32(0x5F3759DF) - lax.shift_right_arithmetic(i, 1)
            y = lax.bitcast_convert_type(i, F32)
            xh = x * 0.5
            y = y * (1.5 - xh * y * y)
            y = y * (1.5 - xh * y * y)
            y = y * (1.5 - xh * y * y)
            node_v[pl.ds(g * 16, 16)] = y
            d2_v[pl.ds(g * 16, 16)] = y * y
            return 0
        lax.fori_loop(0, ns // 16, rsq, 0)
        pltpu.sync_copy(node_v, dinvsh.at[pl.ds(s * ns, ns)])

        @pl.when(c == 0)
        def _():
            pltpu.sync_copy(d2_v, d2_o.at[pl.ds(s * ns, ns)])

        plsc.subcore_barrier()
        pltpu.sync_copy(dinvsh, dinvt)

        def nrow(r, _):
            b = c * hb + r

            def grp(g, _):
                ss = sdb[b, 0, pl.ds(g * 16, 16)]
                dd = sdb[b, 1, pl.ds(g * 16, 16)]
                ww = ewb[b, pl.ds(g * 16, 16)]
                a = plsc.load_gather(dinvt, [ss])
                bb = plsc.load_gather(dinvt, [dd])
                normout[r, pl.ds(g * 16, 16)] = a * ww * bb
                return 0
            lax.fori_loop(0, 8, grp, 0)
            return 0
        lax.fori_loop(0, hb, nrow, 0)
        pltpu.sync_copy(normout, norm_o.at[pl.ds(s * nbt + c * hb, hb)])

    return k


def _sc_scatter(Npad, EP, Wh):
    RW = EP // 128
    nbt = RW // 16
    rpt = Npad // 16
    zc = min(16, rpt)
    nz = rpt // zc
    wc = min(128, rpt)
    nw = rpt // wc
    ng = Wh // 16
    mesh = plsc.VectorSubcoreMesh(core_axis_name="c", subcore_axis_name="s")

    @partial(pl.kernel, mesh=mesh,
             compiler_params=_SC_PARAMS,
             out_type=[jax.ShapeDtypeStruct((Npad, Wh), F32),
                       jax.ShapeDtypeStruct((Npad, Wh), F32)],
             scratch_types=[
                 pltpu.VMEM((nbt, 2, 128), I32),
                 pltpu.VMEM((512,), F32),
                 pltpu.VMEM((4, 128, Wh), F32),
                 pltpu.VMEM((zc, Wh), F32),
                 pltpu.VMEM_SHARED((Npad, Wh), F32),
                 pltpu.SemaphoreType.DMA,
                 pltpu.SemaphoreType.DMA,
                 pltpu.SemaphoreType.DMA,
                 pltpu.SemaphoreType.DMA,
                 pltpu.SemaphoreType.DMA,
                 pltpu.SemaphoreType.DMA,
                 pltpu.SemaphoreType.DMA,
                 pltpu.SemaphoreType.DMA,
                 pltpu.SemaphoreType.DMA,
                 pltpu.SemaphoreType.DMA,
                 pltpu.SemaphoreType.DMA,
                 pltpu.SemaphoreType.DMA,
             ])
    def k(xa_h, xb_h, sd_h, norm_h, oa, ob,
          sdb, normb, rows, zero_v, acc,
          gs0, gs1, gs2, gs3, ss0, ss1, ss2, ss3, ns0, ns1, ns2, ns3):
        c = lax.axis_index("c")
        s = lax.axis_index("s")
        gsem = (gs0, gs1, gs2, gs3)
        ssem = (ss0, ss1, ss2, ss3)
        nsem = (ns0, ns1, ns2, ns3)

        pltpu.sync_copy(sd_h.at[pl.ds(s * nbt, nbt)], sdb)

        def zfill(i, _):
            r = i // ng
            j = i % ng
            zero_v[r, pl.ds(j * 16, 16)] = jnp.zeros((16,), F32)
            return 0
        lax.fori_loop(0, zc * ng, zfill, 0)

        def zcopy(i, _):
            pltpu.sync_copy(zero_v, acc.at[pl.ds(s * rpt + i * zc, zc)])
            return 0
        lax.fori_loop(0, nz, zcopy, 0)
        plsc.subcore_barrier()

        def start_g(b, j):
            @pl.when(c == 0)
            def _():
                pltpu.async_copy(xa_h.at[sdb.at[b, 0]], rows.at[j], gsem[j])

            @pl.when(c == 1)
            def _():
                pltpu.async_copy(xb_h.at[sdb.at[b, 0]], rows.at[j], gsem[j])

        def wait_g(j):
            pltpu.make_async_copy(
                xa_h.at[sdb.at[0, 0]], rows.at[j], gsem[j]).wait()

        def start_n(b, j):
            pltpu.async_copy(
                norm_h.at[pl.ds(s * nbt * 128 + b * 128, 128)],
                normb.at[pl.ds(j * 128, 128)], nsem[j])

        def wait_n(j):
            pltpu.make_async_copy(
                norm_h.at[pl.ds(0, 128)],
                normb.at[pl.ds(j * 128, 128)], nsem[j]).wait()

        def scale(b, j):
            def srow(rr, _):
                for u in range(2):
                    r = rr * 2 + u
                    bc = plsc.load_gather(
                        normb, [jnp.zeros((16,), I32) + j * 128 + r])
                    for g in range(ng):
                        rows[j, r, pl.ds(g * 16, 16)] = (
                            rows[j, r, pl.ds(g * 16, 16)] * bc)
                return 0
            lax.fori_loop(0, 64, srow, 0)

        def start_s(b, j):
            pltpu.async_copy(rows.at[j], acc.at[sdb.at[b, 1]], ssem[j],
                             add=True)

        def wait_s(j):
            pltpu.make_async_copy(
                rows.at[j], acc.at[sdb.at[0, 1]], ssem[j]).wait()

        nloop = nbt // 4
        for j in range(2):
            start_g(j, j)
            start_n(j, j)

        def body(bb, _):
            for j in range(4):
                b = 4 * bb + j
                wait_g(j)
                wait_n(j)
                scale(b, j)
                start_s(b, j)
                jn = (j + 2) % 4
                if j < 2:
                    @pl.when(bb >= 1)
                    def _():
                        wait_s(jn)
                    start_g(b + 2, jn)
                    start_n(b + 2, jn)
                else:
                    wait_s(jn)

                    @pl.when(bb + 1 < nloop)
                    def _():
                        start_g(b + 2, jn)
                        start_n(b + 2, jn)
            return 0
        lax.fori_loop(0, nloop, body, 0)
        wait_s(2)
        wait_s(3)
        plsc.subcore_barrier()

        def wout(i, _):
            r0 = s * rpt + i * wc
            pltpu.sync_copy(acc.at[pl.ds(r0, wc)], rows.at[0, pl.ds(0, wc)])

            @pl.when(c == 0)
            def _():
                pltpu.sync_copy(rows.at[0, pl.ds(0, wc)], oa.at[pl.ds(r0, wc)])

            @pl.when(c == 1)
            def _():
                pltpu.sync_copy(rows.at[0, pl.ds(0, wc)], ob.at[pl.ds(r0, wc)])
            return 0
        lax.fori_loop(0, nw, wout, 0)

    return k


def _tc_layer(N, BN, Din):
    G = N // BN
    Dh = Din // 2

    def body(sa, sb, x, d2, W0, b0, W1a, W1b, W1c, W1d, pa, pb, pc, pd):
        ax = jnp.concatenate([sa[...], sb[...]], axis=1) + d2[...] * x[...]
        t = jnp.maximum(
            jnp.dot(ax, W0[...], preferred_element_type=F32) + b0[...], 0.0)
        pa[...] = jnp.dot(t, W1a[...], preferred_element_type=F32)
        pb[...] = jnp.dot(t, W1b[...], preferred_element_type=F32)
        pc[...] = jnp.dot(t, W1c[...], preferred_element_type=F32)
        pd[...] = jnp.dot(t, W1d[...], preferred_element_type=F32)

    Q = _EMB // 4
    return pl.pallas_call(
        body,
        grid=(G,),
        in_specs=[
            pl.BlockSpec((BN, Dh), lambda i: (i, 0)),
            pl.BlockSpec((BN, Dh), lambda i: (i, 0)),
            pl.BlockSpec((BN, Din), lambda i: (i, 0)),
            pl.BlockSpec((BN, 1), lambda i: (i, 0)),
            pl.BlockSpec((Din, _EMB), lambda i: (0, 0)),
            pl.BlockSpec((1, _EMB), lambda i: (0, 0)),
        ] + [pl.BlockSpec((_EMB, Q), lambda i: (0, 0))] * 4,
        out_specs=[pl.BlockSpec((BN, Q), lambda i: (i, 0))] * 4,
        out_shape=[jax.ShapeDtypeStruct((N, Q), F32)] * 4,
    )


def _tc_mean(N, BN):
    G = N // BN
    Q = _EMB // 4

    def body(s0, s1, s2, s3, p0, p1, p2, p3, d2, b1, out):
        i = pl.program_id(0)
        e = jnp.concatenate([s0[...], s1[...], s2[...], s3[...]], axis=1) \
            + d2[...] * jnp.concatenate(
                [p0[...], p1[...], p2[...], p3[...]], axis=1) + b1[...]
        e = jnp.maximum(e, 0.0)
        ss = jnp.sum(e * e, axis=1, keepdims=True)
        w = 1.0 / jnp.maximum(jnp.sqrt(ss), 1e-12)
        contrib = jnp.sum(w * e, axis=0, keepdims=True)

        @pl.when(i == 0)
        def _():
            out[...] = jnp.zeros_like(out)
        out[...] += contrib

    return pl.pallas_call(
        body,
        grid=(G,),
        in_specs=[pl.BlockSpec((BN, Q), lambda i: (i, 0))] * 8 + [
            pl.BlockSpec((BN, 1), lambda i: (i, 0)),
            pl.BlockSpec((1, _EMB), lambda i: (0, 0)),
        ],
        out_specs=pl.BlockSpec((1, _EMB), lambda i: (0, 0)),
        out_shape=jax.ShapeDtypeStruct((1, _EMB), F32),
    )


def _mish(x):
    return x * jnp.tanh(jax.nn.softplus(x))


def _tc_final(accn, accd, time2, act, Wt1, bt1, Wt2, bt2,
              Wf1, bf1, Wf2, bf2, Wf3, bf3):
    def body(an, ad, tm, ac, wt1, bt1_, wt2, bt2_, wf1, bf1_, wf2, bf2_,
             wf3, bf3_, out):
        hyb = an[...] * (1.0 / _N_NET) + ad[...] * (1.0 / _N_DAG)
        freqs = jnp.exp(lax.broadcasted_iota(I32, (1, 16), 1).astype(F32)
                        * jnp.float32(-np.log(10000.0) / 15.0))
        e = tm[...] * freqs
        pe = jnp.concatenate([jnp.sin(e), jnp.cos(e)], axis=1)
        temb = jnp.dot(_mish(jnp.dot(pe, wt1[...], preferred_element_type=F32)
                             + bt1_[...]),
                       wt2[...], preferred_element_type=F32) + bt2_[...]
        fi = jnp.concatenate([hyb, temb, ac[...]], axis=1)
        z = _mish(jnp.dot(fi, wf1[...], preferred_element_type=F32) + bf1_[...])
        z = jnp.dot(z, wf2[...], preferred_element_type=F32) + bf2_[...]
        out[...] = jnp.dot(z, wf3[...], preferred_element_type=F32) + bf3_[...]

    return pl.pallas_call(
        body,
        out_shape=jax.ShapeDtypeStruct((1, bf3.shape[-1]), F32),
    )(accn, accd, time2, act, Wt1, bt1, Wt2, bt2, Wf1, bf1, Wf2, bf2, Wf3, bf3)


def _pad1(x, n):
    pad = n - x.shape[0]
    return jnp.concatenate([x, jnp.zeros((pad,), x.dtype)])


def _gcn_stack(x, edge_index, edge_weight, W0, b0, W1, b1,
               N, Npad, EP, BN):
    Din = x.shape[1]
    Dh = Din // 2
    src = _pad1(edge_index[0], EP).reshape(EP // 128, 128)
    dst = _pad1(edge_index[1], EP).reshape(EP // 128, 128)
    ew = _pad1(edge_weight, EP).reshape(EP // 128, 128)
    sd = jnp.stack([src, dst], axis=1)

    norm2, d2 = _sc_norm(Npad, EP)(sd, ew)
    norm = norm2.reshape(EP)
    d2c = d2[:N, None]

    xa = x[:, :Dh]
    xb = x[:, Dh:]
    s0a, s0b = _sc_scatter(Npad, EP, Dh)(xa, xb, sd, norm)
    s0a, s0b = s0a[:N], s0b[:N]

    Q = _EMB // 4
    W1q = [W1[:, i * Q:(i + 1) * Q] for i in range(4)]
    p1 = _tc_layer(N, BN, Din)(
        s0a, s0b, x, d2c, W0, b0[None, :], *W1q)

    scat_q = _sc_scatter(Npad, EP, Q)
    s1a, s1b = scat_q(p1[0], p1[1], sd, norm)
    s1c, s1d = scat_q(p1[2], p1[3], sd, norm)
    s1 = [v[:N] for v in (s1a, s1b, s1c, s1d)]

    return _tc_mean(N, BN)(*s1, *p1, d2c, b1[None, :])


def kernel(action, time, net_feature, net_edge_index, net_edge_weight,
           dag_feature, dag_edge_index, dag_edge_weight, batch_size,
           Wn0, bn0, Wn1, bn1, Wd0, bd0, Wd1, bd1,
           Wt1, bt1, Wt2, bt2, Wf1, bf1, Wf2, bf2, Wf3, bf3):
    accn = _gcn_stack(net_feature, net_edge_index, net_edge_weight,
                      Wn0, bn0, Wn1, bn1, _N_NET, _NP_NET, _EP_NET, 1000)
    accd = _gcn_stack(dag_feature, dag_edge_index, dag_edge_weight,
                      Wd0, bd0, Wd1, bd1, _N_DAG, _NP_DAG, _EP_DAG, 1000)
    return _tc_final(accn, accd, time[:, None], action,
                     Wt1, bt1[None, :], Wt2, bt2[None, :],
                     Wf1, bf1[None, :], Wf2, bf2[None, :],
                     Wf3, bf3[None, :])
